# baseline (speedup 1.0000x reference)
import jax
import jax.numpy as jnp
from jax.experimental import pallas as pl
from jax.experimental.pallas import tpu as pltpu


def _mlp_kernel(x_ref, w1a_ref, bv1_ref, w2a_ref, w3a_ref, o_ref):
    h1 = jnp.dot(w1a_ref[...], x_ref[...], preferred_element_type=jnp.float32)
    h1 = jnp.maximum(h1 + bv1_ref[...], 0.0)

    h2 = jnp.dot(w2a_ref[...], h1, preferred_element_type=jnp.float32)
    h2 = jnp.maximum(h2, 0.0)

    y = jnp.dot(w3a_ref[...], h2, preferred_element_type=jnp.float32)
    o_ref[...] = y[0:1, :]


def kernel(x_t, w1, b1, w2, b2, w3, b3):
    in_dim, B = x_t.shape
    hidden = w1.shape[0]

    w1a = jnp.zeros((16, in_dim), jnp.float32)
    w1a = w1a.at[:hidden, :].set(w1)

    bv1 = jnp.zeros((16, 1), jnp.float32)
    bv1 = bv1.at[:hidden, 0].set(b1)
    bv1 = bv1.at[hidden, 0].set(1.0)

    w2a = jnp.zeros((16, 16), jnp.float32)
    w2a = w2a.at[:hidden, :hidden].set(w2)
    w2a = w2a.at[:hidden, hidden].set(b2)
    w2a = w2a.at[hidden, hidden].set(1.0)

    w3a = jnp.zeros((8, 16), jnp.float32)
    w3a = w3a.at[0, :hidden].set(w3[0])
    w3a = w3a.at[0, hidden].set(b3[0])

    tB = 32768
    n_tiles = pl.cdiv(B, tB)
    B_pad = n_tiles * tB
    if B_pad != B:
        x_t = jnp.pad(x_t, ((0, 0), (0, B_pad - B)))

    out = pl.pallas_call(
        _mlp_kernel,
        out_shape=jax.ShapeDtypeStruct((1, B_pad), jnp.float32),
        grid=(n_tiles,),
        in_specs=[
            pl.BlockSpec((in_dim, tB), lambda i: (0, i)),
            pl.BlockSpec((16, in_dim), lambda i: (0, 0)),
            pl.BlockSpec((16, 1), lambda i: (0, 0)),
            pl.BlockSpec((16, 16), lambda i: (0, 0)),
            pl.BlockSpec((8, 16), lambda i: (0, 0)),
        ],
        out_specs=pl.BlockSpec((1, tB), lambda i: (0, i)),
        compiler_params=pltpu.CompilerParams(
            dimension_semantics=("parallel",),
        ),
    )(x_t, w1a, bv1, w2a, w3a)

    return out[0, :B].reshape(B, 1)
